# baseline (speedup 1.0000x reference)
import functools

import jax
import jax.numpy as jnp
from jax import lax
from jax.experimental import pallas as pl
from jax.experimental.pallas import tpu as pltpu
from jax.experimental.pallas import tpu_sc as plsc

N = 10000
D = 768
DP = D // 2
NT = 4
NB = 3

SC_NC = 2
SC_NS = 16
SC_L = 16

DW = D + 128
KCH = 10
CCH = 1024
OUT_ROWS = SC_NC * KCH * CCH
RPT = 64


def _pack2(y):
    u = lax.bitcast_convert_type(y, jnp.uint32)
    r = (u + jnp.uint32(0x8000)) >> jnp.uint32(16)
    return lax.bitcast_convert_type(
        (r[:, DP:] << jnp.uint32(16)) | r[:, :DP], jnp.int32)


def _dense_src_body(h_ref, sWt_ref, lng_ref, lnb_ref, sball_ref, sc_ref,
                    uc_ref, tblA_ref):
    x = h_ref[...].astype(jnp.bfloat16)
    hs = jnp.dot(x, sWt_ref[...], preferred_element_type=jnp.float32)
    mu = jnp.mean(hs, axis=-1, keepdims=True)
    var = jnp.mean((hs - mu) ** 2, axis=-1, keepdims=True)
    ls = (hs - mu) * lax.rsqrt(var + 1e-5) * lng_ref[...] + lnb_ref[...]
    ls = ls.astype(jnp.bfloat16)
    y_all = jnp.dot(ls, sball_ref[...], preferred_element_type=jnp.float32)
    ys = [y_all[:, b * D:(b + 1) * D] for b in range(NB)]
    yu = [y_all[:, (NB + b) * D:(NB + b + 1) * D] for b in range(NB)]
    scc = sc_ref[...]
    ucc = uc_ref[...]
    for t in range(NT):
        s_acc = scc[t, 0] * ys[0] + scc[t, 1] * ys[1] + scc[t, 2] * ys[2]
        u_acc = ucc[t, 0] * yu[0] + ucc[t, 1] * yu[1] + ucc[t, 2] * yu[2]
        tblA_ref[t] = _pack2(jnp.maximum(s_acc, 0.0))
        tblA_ref[NT + t] = _pack2(jnp.maximum(u_acc, 0.0))


def _dense_dst_body(h_ref, dWt_ref, lng_ref, lnb_ref, dball_ref, dc_ref,
                    tblB_ref):
    x = h_ref[...].astype(jnp.bfloat16)
    hd = jnp.dot(x, dWt_ref[...], preferred_element_type=jnp.float32)
    mu = jnp.mean(hd, axis=-1, keepdims=True)
    var = jnp.mean((hd - mu) ** 2, axis=-1, keepdims=True)
    ld = (hd - mu) * lax.rsqrt(var + 1e-5) * lng_ref[...] + lnb_ref[...]
    ld = ld.astype(jnp.bfloat16)
    tblB_ref[0] = _pack2(hd)
    y_all = jnp.dot(ld, dball_ref[...], preferred_element_type=jnp.float32)
    yd = [y_all[:, b * D:(b + 1) * D] for b in range(NB)]
    dcc = dc_ref[...]
    for t in range(NT):
        d_acc = dcc[t, 0] * yd[0] + dcc[t, 1] * yd[1] + dcc[t, 2] * yd[2]
        tblB_ref[1 + t] = _pack2(jnp.maximum(d_acc, 0.0))


def _edge_body(src_ref, dst_ref, et_ref, fe_ref, se_ref, t1_ref, t2_ref,
               he_ref, fq_ref, ph_ref, ct_ref, ia_ref, ib_ref, rr_ref):
    he = he_ref[...]
    t1 = t1_ref[...]
    t2 = t2_ref[...]
    src = src_ref[...]
    dst = dst_ref[...]
    hete = he == 1
    tsel = jnp.where(hete, et_ref[...], jnp.maximum(fe_ref[...], se_ref[...]))
    ct_ref[...] = _pack2(jnp.cos(tsel * fq_ref[...] + ph_ref[...]))
    ia_ref[...] = jnp.where(hete, t1 * N + src, NT * N + t2 * N + src)
    ib_ref[...] = jnp.where(hete, dst, N + t1 * N + dst)
    rr_ref[...] = jnp.where(hete, dst,
                            jnp.where(he == 0, N + dst, jnp.int32(1 << 30)))


def _sc_body(tblA, tblB, ct, ia_hbm, ib_hbm, rr_hbm, out_hbm,
             recA, recB, recR, blist, bufA, bufB, bufC,
             msgb, tgtb, iA, iB, iE, zbuf, chunk, sem1, sem2, sem3, sem4):
    c = lax.axis_index("c")
    s = lax.axis_index("s")
    ept = recR.shape[0]
    base = s * ept
    inv_scale = 1.0 / (768.0 ** 0.5)
    iota = lax.broadcasted_iota(jnp.int32, (SC_L,), 0)
    zero16 = jnp.zeros((SC_L,), jnp.float32)
    unit16 = (iota == 0).astype(jnp.float32)
    MHI = jnp.int32(-65536)
    NP = DW // 128

    pltpu.sync_copy(ia_hbm.at[pl.ds(base, ept)], recA)
    pltpu.sync_copy(ib_hbm.at[pl.ds(base, ept)], recB)
    pltpu.sync_copy(rr_hbm.at[pl.ds(base, ept)], recR)

    def zmsg(k, _):
        for p in range(NP):
            for i in range(128 // SC_L):
                msgb[p, k, pl.ds(i * SC_L, SC_L)] = zero16
        return 0
    lax.fori_loop(0, SC_L, zmsg, 0)

    def zzb(k, _):
        for i in range(128 // SC_L):
            zbuf[k, pl.ds(i * SC_L, SC_L)] = zero16
        return 0
    lax.fori_loop(0, zbuf.shape[0], zzb, 0)

    lo0 = c * KCH * CCH
    row0 = s * RPT
    zrows = zbuf.shape[0]
    nz = RPT * NP // zrows
    for k in range(KCH):
        lo = lo0 + k * CCH

        def bucket(g, cnt):
            rv = recR[pl.ds(g * SC_L, SC_L)]
            m = (rv >= lo) & (rv < lo + CCH)
            plsc.store_compressed(blist.at[pl.ds(cnt, SC_L)],
                                  g * SC_L + iota, mask=m)
            return cnt + jnp.sum(m.astype(jnp.int32))
        n = lax.fori_loop(0, ept // SC_L, bucket, jnp.int32(0))

        def zc(z, _):
            pltpu.sync_copy(zbuf,
                            chunk.at[pl.ds(row0 * NP + z * zrows, zrows)])
            return 0
        lax.fori_loop(0, nz, zc, 0)
        plsc.subcore_barrier()

        ngroups = (n + SC_L - 1) // SC_L

        def group(g, _):
            j = g * SC_L
            bl = blist[pl.ds(j, SC_L)]
            valid = (j + iota) < n
            bl = jnp.where(valid, bl, 0)
            av = plsc.load_gather(recA, [bl])
            bv = plsc.load_gather(recB, [bl])
            rv = plsc.load_gather(recR, [bl])
            iA[...] = av
            iB[...] = bv
            iE[...] = base + bl
            r7 = (rv - lo) * NP
            for p in range(NP):
                tgtb[p] = jnp.where(valid, r7 + p, -1)
            cpA = pltpu.async_copy(tblA.at[iA], bufA, sem1)
            cpB = pltpu.async_copy(tblB.at[iB], bufB, sem2)
            cpC = pltpu.async_copy(ct.at[iE], bufC, sem3)
            cpA.wait()
            cpB.wait()
            cpC.wait()

            def row(kk, _):
                acc = zero16
                for i in range(DP // SC_L):
                    sl = pl.ds(i * SC_L, SC_L)
                    wC = bufC[kk, sl]
                    wA = bufA[kk, sl]
                    wB = bufB[kk, sl]
                    mlo = (lax.bitcast_convert_type(wC << 16, jnp.float32)
                           + lax.bitcast_convert_type(wA << 16, jnp.float32))
                    mhi = (lax.bitcast_convert_type(wC & MHI, jnp.float32)
                           + lax.bitcast_convert_type(wA & MHI, jnp.float32))
                    msgb[i // 8, kk, pl.ds((i % 8) * SC_L, SC_L)] = mlo
                    msgb[3 + i // 8, kk, pl.ds((i % 8) * SC_L, SC_L)] = mhi
                    acc = (acc
                           + mlo * lax.bitcast_convert_type(wB << 16,
                                                            jnp.float32)
                           + mhi * lax.bitcast_convert_type(wB & MHI,
                                                            jnp.float32))
                e = jnp.sum(acc) * inv_scale
                pf = jnp.where(j + kk < n, 1.0, 0.0)
                p16 = jnp.exp(jnp.full((SC_L,), e, jnp.float32)) * pf
                msgb[D // 128, kk, pl.ds(0, SC_L)] = unit16
                for i in range(D // SC_L + 1):
                    sl = pl.ds((i % 8) * SC_L, SC_L)
                    b = msgb.at[i // 8]
                    b[kk, sl] = b[kk, sl] * p16
                return 0

            lax.fori_loop(0, SC_L, row, 0)
            cps = [
                pltpu.async_copy(
                    msgb.at[p],
                    chunk.at[plsc.Indices(tgtb.at[p], ignored_value=-1)],
                    sem4, add=True)
                for p in range(NP)
            ]
            for cp in cps:
                cp.wait()
            return 0

        lax.fori_loop(0, ngroups, group, 0)
        plsc.subcore_barrier()

        def fl(z, _):
            pltpu.sync_copy(chunk.at[pl.ds(row0 * NP + z * zrows, zrows)],
                            out_hbm.at[pl.ds((lo + row0) * NP + z * zrows,
                                             zrows)])
            return 0
        lax.fori_loop(0, nz, fl, 0)


def _final_body(hh_ref, ho_ref, w1_ref, w2_ref, lb_ref, out_ref):
    a = hh_ref[...]
    b = ho_ref[...]
    hh = a[:, :D] / jnp.maximum(a[:, D:D + 1], 1e-30)
    ho = b[:, :D] / jnp.maximum(b[:, D:D + 1], 1e-30)
    z = (jnp.dot(hh.astype(jnp.bfloat16), w1_ref[...],
                 preferred_element_type=jnp.float32)
         + jnp.dot(ho.astype(jnp.bfloat16), w2_ref[...],
                   preferred_element_type=jnp.float32)
         + lb_ref[...])
    z = jnp.maximum(z, 0.0)
    nrm = jnp.sqrt(jnp.sum(z * z, axis=1, keepdims=True))
    nrm = jnp.where(nrm == 0.0, 1.0, nrm)
    out_ref[...] = z / nrm


def kernel(h, edge_src, edge_dst, etime, first_etime, second_etime,
           first_etype, second_etype, hete, src_W, dst_W, ln_g, ln_b,
           basis_freq, phase, src_basis, src_coef, dst_basis, dst_coef,
           user_basis, user_coef, lin_W, lin_b):
    E = edge_src.shape[0]
    ept = ((E + SC_NS * SC_L - 1) // (SC_NS * SC_L)) * SC_L
    E_pad = ept * SC_NS

    BLK = 400
    grid_a = N // BLK
    full = lambda shp: pl.BlockSpec(shp, lambda i: tuple(0 for _ in shp))
    tblA = pl.pallas_call(
        _dense_src_body,
        grid=(grid_a,),
        in_specs=[
            pl.BlockSpec((BLK, D), lambda i: (i, 0)),
            full((D, D)),
            full((1, D)), full((1, D)),
            full((D, 2 * NB * D)), full((NT, NB)),
            full((NT, NB)),
        ],
        out_specs=pl.BlockSpec((2 * NT, BLK, DP), lambda i: (0, i, 0)),
        out_shape=jax.ShapeDtypeStruct((2 * NT, N, DP), jnp.int32),
    )(h, src_W.T.astype(jnp.bfloat16), ln_g.reshape(1, D),
      ln_b.reshape(1, D),
      jnp.concatenate([src_basis[b] for b in range(NB)]
                      + [user_basis[b] for b in range(NB)],
                      axis=1).astype(jnp.bfloat16),
      src_coef, user_coef)
    tblB = pl.pallas_call(
        _dense_dst_body,
        grid=(grid_a,),
        in_specs=[
            pl.BlockSpec((BLK, D), lambda i: (i, 0)),
            full((D, D)),
            full((1, D)), full((1, D)),
            full((D, NB * D)), full((NT, NB)),
        ],
        out_specs=pl.BlockSpec((1 + NT, BLK, DP), lambda i: (0, i, 0)),
        out_shape=jax.ShapeDtypeStruct((1 + NT, N, DP), jnp.int32),
    )(h, dst_W.T.astype(jnp.bfloat16), ln_g.reshape(1, D),
      ln_b.reshape(1, D),
      jnp.concatenate([dst_basis[b] for b in range(NB)],
                      axis=1).astype(jnp.bfloat16),
      dst_coef)
    tblA = tblA.reshape(2 * NT * N, DP)
    tblB = tblB.reshape((1 + NT) * N, DP)

    def pad1(x, val):
        return jnp.pad(x, (0, E_pad - E), constant_values=val).reshape(E_pad, 1)

    EB = 1792
    assert E_pad % EB == 0
    ct, ia, ib, rr = pl.pallas_call(
        _edge_body,
        grid=(E_pad // EB,),
        in_specs=[pl.BlockSpec((EB, 1), lambda i: (i, 0))] * 8
        + [full((1, D)), full((1, D))],
        out_specs=[pl.BlockSpec((EB, DP), lambda i: (i, 0))]
        + [pl.BlockSpec((EB, 1), lambda i: (i, 0))] * 3,
        out_shape=[
            jax.ShapeDtypeStruct((E_pad, DP), jnp.int32),
            jax.ShapeDtypeStruct((E_pad, 1), jnp.int32),
            jax.ShapeDtypeStruct((E_pad, 1), jnp.int32),
            jax.ShapeDtypeStruct((E_pad, 1), jnp.int32),
        ],
    )(pad1(edge_src, 0), pad1(edge_dst, 0), pad1(etime, 0.0),
      pad1(first_etime, 0.0), pad1(second_etime, 0.0),
      pad1(first_etype, 0), pad1(second_etype, 0), pad1(hete, 2),
      basis_freq.reshape(1, D), phase.reshape(1, D))
    ia = ia.reshape(E_pad)
    ib = ib.reshape(E_pad)
    rr = rr.reshape(E_pad)

    mesh = plsc.VectorSubcoreMesh(core_axis_name="c", subcore_axis_name="s",
                                  num_cores=SC_NC, num_subcores=SC_NS)
    acc = pl.kernel(
        _sc_body,
        out_type=jax.ShapeDtypeStruct((OUT_ROWS * (DW // 128), 128), jnp.float32),
        mesh=mesh,
        compiler_params=pltpu.CompilerParams(needs_layout_passes=False),
        scratch_types=[
            pltpu.VMEM((ept,), jnp.int32),
            pltpu.VMEM((ept,), jnp.int32),
            pltpu.VMEM((ept,), jnp.int32),
            pltpu.VMEM((ept + SC_L,), jnp.int32),
            pltpu.VMEM((SC_L, DP), jnp.int32),
            pltpu.VMEM((SC_L, DP), jnp.int32),
            pltpu.VMEM((SC_L, DP), jnp.int32),
            pltpu.VMEM((7, SC_L, 128), jnp.float32),
            pltpu.VMEM((7, SC_L), jnp.int32),
            pltpu.VMEM((SC_L,), jnp.int32),
            pltpu.VMEM((SC_L,), jnp.int32),
            pltpu.VMEM((SC_L,), jnp.int32),
            pltpu.VMEM((56, 128), jnp.float32),
            pltpu.VMEM_SHARED((CCH * (DW // 128), 128), jnp.float32),
            pltpu.SemaphoreType.DMA,
            pltpu.SemaphoreType.DMA,
            pltpu.SemaphoreType.DMA,
            pltpu.SemaphoreType.DMA,
        ],
    )(tblA, tblB, ct, ia, ib, rr)
    acc = acc.reshape(OUT_ROWS, DW)

    BC = 400
    lin_Wt = lin_W.T.astype(jnp.bfloat16)
    z = pl.pallas_call(
        _final_body,
        grid=(N // BC,),
        in_specs=[
            pl.BlockSpec((BC, DW), lambda i: (i, 0)),
            pl.BlockSpec((BC, DW), lambda i: (i + N // BC, 0)),
            full((D, D)), full((D, D)), full((1, D)),
        ],
        out_specs=pl.BlockSpec((BC, D), lambda i: (i, 0)),
        out_shape=jax.ShapeDtypeStruct((N, D), jnp.float32),
    )(acc, acc, lin_Wt[:D], lin_Wt[D:], lin_b.reshape(1, D))
    return z


# speedup vs baseline: 1.0080x; 1.0080x over previous
import functools

import jax
import jax.numpy as jnp
from jax import lax
from jax.experimental import pallas as pl
from jax.experimental.pallas import tpu as pltpu
from jax.experimental.pallas import tpu_sc as plsc

N = 10000
D = 768
DP = D // 2
NT = 4
NB = 3

SC_NC = 2
SC_NS = 16
SC_L = 16

DW = D + 128
KCH = 10
CCH = 1024
OUT_ROWS = SC_NC * KCH * CCH
RPT = 64


def _pack2(y):
    u = lax.bitcast_convert_type(y, jnp.uint32)
    r = (u + jnp.uint32(0x8000)) >> jnp.uint32(16)
    return lax.bitcast_convert_type(
        (r[:, DP:] << jnp.uint32(16)) | r[:, :DP], jnp.int32)


def _dense_src_body(h_ref, sWt_ref, lng_ref, lnb_ref, sb_ref, sc_ref,
                    ub_ref, uc_ref, tblA_ref):
    x = h_ref[...].astype(jnp.bfloat16)
    hs = jnp.dot(x, sWt_ref[...], preferred_element_type=jnp.float32)
    mu = jnp.mean(hs, axis=-1, keepdims=True)
    var = jnp.mean((hs - mu) ** 2, axis=-1, keepdims=True)
    ls = (hs - mu) * lax.rsqrt(var + 1e-5) * lng_ref[...] + lnb_ref[...]
    ls = ls.astype(jnp.bfloat16)
    ys = [jnp.dot(ls, sb_ref[b], preferred_element_type=jnp.float32)
          for b in range(NB)]
    yu = [jnp.dot(ls, ub_ref[b], preferred_element_type=jnp.float32)
          for b in range(NB)]
    scc = sc_ref[...]
    ucc = uc_ref[...]
    for t in range(NT):
        s_acc = scc[t, 0] * ys[0] + scc[t, 1] * ys[1] + scc[t, 2] * ys[2]
        u_acc = ucc[t, 0] * yu[0] + ucc[t, 1] * yu[1] + ucc[t, 2] * yu[2]
        tblA_ref[t] = _pack2(jnp.maximum(s_acc, 0.0))
        tblA_ref[NT + t] = _pack2(jnp.maximum(u_acc, 0.0))


def _dense_dst_body(h_ref, dWt_ref, lng_ref, lnb_ref, db_ref, dc_ref,
                    tblB_ref):
    x = h_ref[...].astype(jnp.bfloat16)
    hd = jnp.dot(x, dWt_ref[...], preferred_element_type=jnp.float32)
    mu = jnp.mean(hd, axis=-1, keepdims=True)
    var = jnp.mean((hd - mu) ** 2, axis=-1, keepdims=True)
    ld = (hd - mu) * lax.rsqrt(var + 1e-5) * lng_ref[...] + lnb_ref[...]
    ld = ld.astype(jnp.bfloat16)
    tblB_ref[0] = _pack2(hd)
    yd = [jnp.dot(ld, db_ref[b], preferred_element_type=jnp.float32)
          for b in range(NB)]
    dcc = dc_ref[...]
    for t in range(NT):
        d_acc = dcc[t, 0] * yd[0] + dcc[t, 1] * yd[1] + dcc[t, 2] * yd[2]
        tblB_ref[1 + t] = _pack2(jnp.maximum(d_acc, 0.0))


def _edge_body(src_ref, dst_ref, et_ref, fe_ref, se_ref, t1_ref, t2_ref,
               he_ref, fq_ref, ph_ref, ct_ref, ia_ref, ib_ref, rr_ref):
    he = he_ref[...]
    t1 = t1_ref[...]
    t2 = t2_ref[...]
    src = src_ref[...]
    dst = dst_ref[...]
    hete = he == 1
    tsel = jnp.where(hete, et_ref[...], jnp.maximum(fe_ref[...], se_ref[...]))
    ct_ref[...] = _pack2(jnp.cos(tsel * fq_ref[...] + ph_ref[...]))
    ia_ref[...] = jnp.where(hete, t1 * N + src, NT * N + t2 * N + src)
    ib_ref[...] = jnp.where(hete, dst, N + t1 * N + dst)
    rr_ref[...] = jnp.where(hete, dst,
                            jnp.where(he == 0, N + dst, jnp.int32(1 << 30)))


def _sc_body(tblA, tblB, ct, ia_hbm, ib_hbm, rr_hbm, out_hbm,
             recA, recB, recR, blist, bufA, bufB, bufC,
             msgb, tgtb, iA, iB, iE, zbuf, chunk, sem1, sem2, sem3, sem4):
    c = lax.axis_index("c")
    s = lax.axis_index("s")
    ept = recR.shape[0]
    base = s * ept
    inv_scale = 1.0 / (768.0 ** 0.5)
    iota = lax.broadcasted_iota(jnp.int32, (SC_L,), 0)
    zero16 = jnp.zeros((SC_L,), jnp.float32)
    unit16 = (iota == 0).astype(jnp.float32)
    MHI = jnp.int32(-65536)
    NP = DW // 128

    pltpu.sync_copy(ia_hbm.at[pl.ds(base, ept)], recA)
    pltpu.sync_copy(ib_hbm.at[pl.ds(base, ept)], recB)
    pltpu.sync_copy(rr_hbm.at[pl.ds(base, ept)], recR)

    def zmsg(k, _):
        for p in range(NP):
            for i in range(128 // SC_L):
                msgb[p, k, pl.ds(i * SC_L, SC_L)] = zero16
        return 0
    lax.fori_loop(0, SC_L, zmsg, 0)

    def zzb(k, _):
        for i in range(128 // SC_L):
            zbuf[k, pl.ds(i * SC_L, SC_L)] = zero16
        return 0
    lax.fori_loop(0, zbuf.shape[0], zzb, 0)

    lo0 = c * KCH * CCH
    row0 = s * RPT
    zrows = zbuf.shape[0]
    nz = RPT * NP // zrows
    for k in range(KCH):
        lo = lo0 + k * CCH

        def bucket(g, cnt):
            rv = recR[pl.ds(g * SC_L, SC_L)]
            m = (rv >= lo) & (rv < lo + CCH)
            plsc.store_compressed(blist.at[pl.ds(cnt, SC_L)],
                                  g * SC_L + iota, mask=m)
            return cnt + jnp.sum(m.astype(jnp.int32))
        n = lax.fori_loop(0, ept // SC_L, bucket, jnp.int32(0))

        def zc(z, _):
            pltpu.sync_copy(zbuf,
                            chunk.at[pl.ds(row0 * NP + z * zrows, zrows)])
            return 0
        lax.fori_loop(0, nz, zc, 0)
        plsc.subcore_barrier()

        ngroups = (n + SC_L - 1) // SC_L

        def group(g, _):
            j = g * SC_L
            bl = blist[pl.ds(j, SC_L)]
            valid = (j + iota) < n
            bl = jnp.where(valid, bl, 0)
            av = plsc.load_gather(recA, [bl])
            bv = plsc.load_gather(recB, [bl])
            rv = plsc.load_gather(recR, [bl])
            iA[...] = av
            iB[...] = bv
            iE[...] = base + bl
            r7 = (rv - lo) * NP
            for p in range(NP):
                tgtb[p] = jnp.where(valid, r7 + p, -1)
            cpA = pltpu.async_copy(tblA.at[iA], bufA, sem1)
            cpB = pltpu.async_copy(tblB.at[iB], bufB, sem2)
            cpC = pltpu.async_copy(ct.at[iE], bufC, sem3)
            cpA.wait()
            cpB.wait()
            cpC.wait()

            def row(kk, _):
                acc = zero16
                for i in range(DP // SC_L):
                    sl = pl.ds(i * SC_L, SC_L)
                    wC = bufC[kk, sl]
                    wA = bufA[kk, sl]
                    wB = bufB[kk, sl]
                    mlo = (lax.bitcast_convert_type(wC << 16, jnp.float32)
                           + lax.bitcast_convert_type(wA << 16, jnp.float32))
                    mhi = (lax.bitcast_convert_type(wC & MHI, jnp.float32)
                           + lax.bitcast_convert_type(wA & MHI, jnp.float32))
                    msgb[i // 8, kk, pl.ds((i % 8) * SC_L, SC_L)] = mlo
                    msgb[3 + i // 8, kk, pl.ds((i % 8) * SC_L, SC_L)] = mhi
                    acc = (acc
                           + mlo * lax.bitcast_convert_type(wB << 16,
                                                            jnp.float32)
                           + mhi * lax.bitcast_convert_type(wB & MHI,
                                                            jnp.float32))
                e = jnp.sum(acc) * inv_scale
                pf = jnp.where(j + kk < n, 1.0, 0.0)
                p16 = jnp.exp(jnp.full((SC_L,), e, jnp.float32)) * pf
                msgb[D // 128, kk, pl.ds(0, SC_L)] = unit16
                for i in range(D // SC_L + 1):
                    sl = pl.ds((i % 8) * SC_L, SC_L)
                    b = msgb.at[i // 8]
                    b[kk, sl] = b[kk, sl] * p16
                return 0

            lax.fori_loop(0, SC_L, row, 0)
            cps = [
                pltpu.async_copy(
                    msgb.at[p],
                    chunk.at[plsc.Indices(tgtb.at[p], ignored_value=-1)],
                    sem4, add=True)
                for p in range(NP)
            ]
            for cp in cps:
                cp.wait()
            return 0

        lax.fori_loop(0, ngroups, group, 0)
        plsc.subcore_barrier()

        def fl(z, _):
            pltpu.sync_copy(chunk.at[pl.ds(row0 * NP + z * zrows, zrows)],
                            out_hbm.at[pl.ds((lo + row0) * NP + z * zrows,
                                             zrows)])
            return 0
        lax.fori_loop(0, nz, fl, 0)


def _final_body(hh_ref, ho_ref, w1_ref, w2_ref, lb_ref, out_ref):
    a = hh_ref[...]
    b = ho_ref[...]
    hh = a[:, :D] / jnp.maximum(a[:, D:D + 1], 1e-30)
    ho = b[:, :D] / jnp.maximum(b[:, D:D + 1], 1e-30)
    z = (jnp.dot(hh.astype(jnp.bfloat16), w1_ref[...],
                 preferred_element_type=jnp.float32)
         + jnp.dot(ho.astype(jnp.bfloat16), w2_ref[...],
                   preferred_element_type=jnp.float32)
         + lb_ref[...])
    z = jnp.maximum(z, 0.0)
    nrm = jnp.sqrt(jnp.sum(z * z, axis=1, keepdims=True))
    nrm = jnp.where(nrm == 0.0, 1.0, nrm)
    out_ref[...] = z / nrm


def kernel(h, edge_src, edge_dst, etime, first_etime, second_etime,
           first_etype, second_etype, hete, src_W, dst_W, ln_g, ln_b,
           basis_freq, phase, src_basis, src_coef, dst_basis, dst_coef,
           user_basis, user_coef, lin_W, lin_b):
    E = edge_src.shape[0]
    ept = ((E + SC_NS * SC_L - 1) // (SC_NS * SC_L)) * SC_L
    E_pad = ept * SC_NS

    BLK = 400
    grid_a = N // BLK
    full = lambda shp: pl.BlockSpec(shp, lambda i: tuple(0 for _ in shp))
    tblA = pl.pallas_call(
        _dense_src_body,
        grid=(grid_a,),
        in_specs=[
            pl.BlockSpec((BLK, D), lambda i: (i, 0)),
            full((D, D)),
            full((1, D)), full((1, D)),
            full((NB, D, D)), full((NT, NB)),
            full((NB, D, D)), full((NT, NB)),
        ],
        out_specs=pl.BlockSpec((2 * NT, BLK, DP), lambda i: (0, i, 0)),
        out_shape=jax.ShapeDtypeStruct((2 * NT, N, DP), jnp.int32),
    )(h, src_W.T.astype(jnp.bfloat16), ln_g.reshape(1, D),
      ln_b.reshape(1, D), src_basis.astype(jnp.bfloat16), src_coef,
      user_basis.astype(jnp.bfloat16), user_coef)
    tblB = pl.pallas_call(
        _dense_dst_body,
        grid=(grid_a,),
        in_specs=[
            pl.BlockSpec((BLK, D), lambda i: (i, 0)),
            full((D, D)),
            full((1, D)), full((1, D)),
            full((NB, D, D)), full((NT, NB)),
        ],
        out_specs=pl.BlockSpec((1 + NT, BLK, DP), lambda i: (0, i, 0)),
        out_shape=jax.ShapeDtypeStruct((1 + NT, N, DP), jnp.int32),
    )(h, dst_W.T.astype(jnp.bfloat16), ln_g.reshape(1, D),
      ln_b.reshape(1, D), dst_basis.astype(jnp.bfloat16), dst_coef)
    tblA = tblA.reshape(2 * NT * N, DP)
    tblB = tblB.reshape((1 + NT) * N, DP)

    def pad1(x, val):
        return jnp.pad(x, (0, E_pad - E), constant_values=val).reshape(E_pad, 1)

    EB = 1792
    assert E_pad % EB == 0
    ct, ia, ib, rr = pl.pallas_call(
        _edge_body,
        grid=(E_pad // EB,),
        in_specs=[pl.BlockSpec((EB, 1), lambda i: (i, 0))] * 8
        + [full((1, D)), full((1, D))],
        out_specs=[pl.BlockSpec((EB, DP), lambda i: (i, 0))]
        + [pl.BlockSpec((EB, 1), lambda i: (i, 0))] * 3,
        out_shape=[
            jax.ShapeDtypeStruct((E_pad, DP), jnp.int32),
            jax.ShapeDtypeStruct((E_pad, 1), jnp.int32),
            jax.ShapeDtypeStruct((E_pad, 1), jnp.int32),
            jax.ShapeDtypeStruct((E_pad, 1), jnp.int32),
        ],
    )(pad1(edge_src, 0), pad1(edge_dst, 0), pad1(etime, 0.0),
      pad1(first_etime, 0.0), pad1(second_etime, 0.0),
      pad1(first_etype, 0), pad1(second_etype, 0), pad1(hete, 2),
      basis_freq.reshape(1, D), phase.reshape(1, D))
    ia = ia.reshape(E_pad)
    ib = ib.reshape(E_pad)
    rr = rr.reshape(E_pad)

    mesh = plsc.VectorSubcoreMesh(core_axis_name="c", subcore_axis_name="s",
                                  num_cores=SC_NC, num_subcores=SC_NS)
    acc = pl.kernel(
        _sc_body,
        out_type=jax.ShapeDtypeStruct((OUT_ROWS * (DW // 128), 128), jnp.float32),
        mesh=mesh,
        compiler_params=pltpu.CompilerParams(needs_layout_passes=False),
        scratch_types=[
            pltpu.VMEM((ept,), jnp.int32),
            pltpu.VMEM((ept,), jnp.int32),
            pltpu.VMEM((ept,), jnp.int32),
            pltpu.VMEM((ept + SC_L,), jnp.int32),
            pltpu.VMEM((SC_L, DP), jnp.int32),
            pltpu.VMEM((SC_L, DP), jnp.int32),
            pltpu.VMEM((SC_L, DP), jnp.int32),
            pltpu.VMEM((7, SC_L, 128), jnp.float32),
            pltpu.VMEM((7, SC_L), jnp.int32),
            pltpu.VMEM((SC_L,), jnp.int32),
            pltpu.VMEM((SC_L,), jnp.int32),
            pltpu.VMEM((SC_L,), jnp.int32),
            pltpu.VMEM((56, 128), jnp.float32),
            pltpu.VMEM_SHARED((CCH * (DW // 128), 128), jnp.float32),
            pltpu.SemaphoreType.DMA,
            pltpu.SemaphoreType.DMA,
            pltpu.SemaphoreType.DMA,
            pltpu.SemaphoreType.DMA,
        ],
    )(tblA, tblB, ct, ia, ib, rr)
    acc = acc.reshape(OUT_ROWS, DW)

    BC = 400
    lin_Wt = lin_W.T.astype(jnp.bfloat16)
    z = pl.pallas_call(
        _final_body,
        grid=(N // BC,),
        in_specs=[
            pl.BlockSpec((BC, DW), lambda i: (i, 0)),
            pl.BlockSpec((BC, DW), lambda i: (i + N // BC, 0)),
            full((D, D)), full((D, D)), full((1, D)),
        ],
        out_specs=pl.BlockSpec((BC, D), lambda i: (i, 0)),
        out_shape=jax.ShapeDtypeStruct((N, D), jnp.float32),
    )(acc, acc, lin_Wt[:D], lin_Wt[D:], lin_b.reshape(1, D))
    return z
